# Initial kernel scaffold; baseline (speedup 1.0000x reference)
import jax
import jax.numpy as jnp
from jax.experimental import pallas as pl


def kernel(f00, f01, f02, f03, f04, f05, f06, f07, f08, f09, f10, f11, f12, f13, f14, f15, f16, f17, f18, f19, f20, f21, f22, f23, f24, f25, tbl_f00, tbl_f01, tbl_f02, tbl_f03, tbl_f04, tbl_f05, tbl_f06, tbl_f07, tbl_f08, tbl_f09, tbl_f10, tbl_f11, tbl_f12, tbl_f13, tbl_f14, tbl_f15, tbl_f16, tbl_f17, tbl_f18, tbl_f19, tbl_f20, tbl_f21, tbl_f22, tbl_f23, tbl_f24, tbl_f25, W, b, gamma, beta):
    raise NotImplementedError("write your pallas kernel here")


# speedup vs baseline: 3.7057x; 3.7057x over previous
import functools

import jax
import jax.numpy as jnp
from jax import lax
from jax.experimental import pallas as pl
from jax.experimental.pallas import tpu as pltpu
from jax.experimental.pallas import tpu_sc as plsc

NF = 26
VOCAB = 100000
D = 128
B, L = 1024, 50
N = B * L
NC, NS = 2, 16
NW = NC * NS
TPW = N // NW
C = 80
NCHUNK = TPW // C


def _sc_body(idx_hbm, *rest):
    tbls = rest[:NF]
    out_hbm = rest[NF]
    idx_v, acc_v, s0, s1, sem_a, sem0, sem1 = rest[NF + 1:]

    wid = lax.axis_index("s") * NC + lax.axis_index("c")
    base = wid * TPW
    pltpu.sync_copy(idx_hbm.at[wid], idx_v)

    stg = [s0, s1]
    sems = [sem0, sem1]

    def chunk_body(c, carry):
        off = pl.multiple_of(c * C, C)

        def idx_sl(f):
            return idx_v.at[pl.ds(f * TPW + off, C)]

        def accum(stg_ref):
            def tok(t, carry2):
                for j in range(8):
                    sl = pl.ds(j * 16, 16)
                    plsc.addupdate(acc_v.at[t, sl], stg_ref[t, sl])
                return carry2
            lax.fori_loop(0, C, tok, 0, unroll=2)

        cps = [None, None]
        ha = pltpu.async_copy(tbls[0].at[idx_sl(0)], acc_v, sem_a)
        cps[0] = pltpu.async_copy(tbls[1].at[idx_sl(1)], stg[0], sems[0])
        ha.wait()
        for f in range(1, NF):
            cur = (f - 1) % 2
            nxt = f % 2
            if f + 1 < NF:
                cps[nxt] = pltpu.async_copy(
                    tbls[f + 1].at[idx_sl(f + 1)], stg[nxt], sems[nxt])
            cps[cur].wait()
            accum(stg[cur])
        pltpu.sync_copy(acc_v, out_hbm.at[pl.ds(base + off, C)])
        return carry

    lax.fori_loop(0, NCHUNK, chunk_body, 0)


@functools.partial(jax.jit, static_argnums=())
def _gather_sum(idx_all, *tbls):
    mesh = plsc.VectorSubcoreMesh(core_axis_name="c", subcore_axis_name="s")
    return pl.kernel(
        _sc_body,
        mesh=mesh,
        out_type=jax.ShapeDtypeStruct((N, D), jnp.float32),
        scratch_types=[
            pltpu.VMEM((NF * TPW,), jnp.int32),
            pltpu.VMEM((C, D), jnp.float32),
            pltpu.VMEM((C, D), jnp.float32),
            pltpu.VMEM((C, D), jnp.float32),
            pltpu.SemaphoreType.DMA,
            pltpu.SemaphoreType.DMA,
            pltpu.SemaphoreType.DMA,
        ],
    )(idx_all, *tbls)


RB = 2048


def _tc_body(x_ref, w_ref, b_ref, g_ref, bt_ref, o_ref):
    x = x_ref[...]
    h = lax.dot_general(x, w_ref[...], (((1,), (1,)), ((), ())),
                        preferred_element_type=jnp.float32)
    h = h + b_ref[...]
    mean = jnp.mean(h, axis=1, keepdims=True)
    hc = h - mean
    var = jnp.mean(hc * hc, axis=1, keepdims=True)
    o_ref[...] = hc * lax.rsqrt(var + 1e-5) * g_ref[...] + bt_ref[...]


def _proj_norm(combined, W, b, gamma, beta):
    return pl.pallas_call(
        _tc_body,
        grid=(N // RB,),
        in_specs=[
            pl.BlockSpec((RB, D), lambda i: (i, 0)),
            pl.BlockSpec((D, D), lambda i: (0, 0)),
            pl.BlockSpec((1, D), lambda i: (0, 0)),
            pl.BlockSpec((1, D), lambda i: (0, 0)),
            pl.BlockSpec((1, D), lambda i: (0, 0)),
        ],
        out_specs=pl.BlockSpec((RB, D), lambda i: (i, 0)),
        out_shape=jax.ShapeDtypeStruct((N, D), jnp.float32),
    )(combined, W, b.reshape(1, D), gamma.reshape(1, D), beta.reshape(1, D))


def kernel(f00, f01, f02, f03, f04, f05, f06, f07, f08, f09, f10, f11, f12,
           f13, f14, f15, f16, f17, f18, f19, f20, f21, f22, f23, f24, f25,
           tbl_f00, tbl_f01, tbl_f02, tbl_f03, tbl_f04, tbl_f05, tbl_f06,
           tbl_f07, tbl_f08, tbl_f09, tbl_f10, tbl_f11, tbl_f12, tbl_f13,
           tbl_f14, tbl_f15, tbl_f16, tbl_f17, tbl_f18, tbl_f19, tbl_f20,
           tbl_f21, tbl_f22, tbl_f23, tbl_f24, tbl_f25,
           W, b, gamma, beta):
    idxs = [f00, f01, f02, f03, f04, f05, f06, f07, f08, f09, f10, f11, f12,
            f13, f14, f15, f16, f17, f18, f19, f20, f21, f22, f23, f24, f25]
    tbls = [tbl_f00, tbl_f01, tbl_f02, tbl_f03, tbl_f04, tbl_f05, tbl_f06,
            tbl_f07, tbl_f08, tbl_f09, tbl_f10, tbl_f11, tbl_f12, tbl_f13,
            tbl_f14, tbl_f15, tbl_f16, tbl_f17, tbl_f18, tbl_f19, tbl_f20,
            tbl_f21, tbl_f22, tbl_f23, tbl_f24, tbl_f25]
    idx_all = (jnp.stack(idxs).reshape(NF, NW, TPW)
               .transpose(1, 0, 2)
               .reshape(NW, NF * TPW))
    combined = _gather_sum(idx_all, *tbls)
    y = _proj_norm(combined, W, b, gamma, beta)
    return y.reshape(B, L, D)
